# baseline (speedup 1.0000x reference)
import jax
import jax.numpy as jnp
from jax.experimental import pallas as pl
from jax.experimental.pallas import tpu as pltpu

NUM_OPS = 8
TM = 1024
TN = 256


def _body(x_ref, mask_ref, alphas_ref, w_ref, b_ref, o_ref):
    a = alphas_ref[...]
    a = a - jnp.max(a)
    e = jnp.exp(a)
    p = e / jnp.sum(e)
    lane = jax.lax.broadcasted_iota(jnp.int32, (1, NUM_OPS), 1)

    xm = (x_ref[...] * mask_ref[...].astype(jnp.float32)).astype(jnp.bfloat16)

    total = None
    for i in range(NUM_OPS):
        p_i = jnp.sum(jnp.where(lane == i, p, 0.0))
        acc = jnp.dot(xm, w_ref[i].astype(jnp.bfloat16),
                      preferred_element_type=jnp.float32)
        val = jnp.maximum(acc * p_i + p_i * b_ref[i], 0.0)
        total = val if total is None else total + val
    o_ref[...] = total


@jax.jit
def kernel(x, mask, alphas, W, b):
    n_tok, d_model = x.shape
    num_ops = W.shape[0]
    mask2d = mask.reshape(n_tok, 1)
    alphas2d = alphas.reshape(1, num_ops)
    b3d = b.reshape(num_ops, 1, d_model)

    grid = (d_model // TN, n_tok // TM)
    out = pl.pallas_call(
        _body,
        grid=grid,
        in_specs=[
            pl.BlockSpec((TM, d_model), lambda n, m: (m, 0)),
            pl.BlockSpec((TM, 1), lambda n, m: (m, 0)),
            pl.BlockSpec((1, num_ops), lambda n, m: (0, 0)),
            pl.BlockSpec((num_ops, d_model, TN), lambda n, m: (0, 0, n)),
            pl.BlockSpec((num_ops, 1, TN), lambda n, m: (0, 0, n)),
        ],
        out_specs=pl.BlockSpec((TM, TN), lambda n, m: (m, n)),
        out_shape=jax.ShapeDtypeStruct((n_tok, d_model), jnp.float32),
        compiler_params=pltpu.CompilerParams(
            dimension_semantics=("arbitrary", "arbitrary"),
        ),
    )(x, mask2d, alphas2d, W, b3d)
    return out
